# baseline (speedup 1.0000x reference)
import functools

import jax
import jax.numpy as jnp
from jax import lax
from jax.experimental import pallas as pl
from jax.experimental.pallas import tpu as pltpu
from jax.experimental.pallas import tpu_sc as plsc

EMB = 16
BATCH = 16384
NUM_CORES = 2
NUM_SUBCORES = 16
NW = NUM_CORES * NUM_SUBCORES
BPW = BATCH // NW
CHUNK = 128
NCHUNK = BPW // CHUNK


def _sc_interac(x0, x1, emb1, emb2):
  mesh = plsc.VectorSubcoreMesh(core_axis_name="c", subcore_axis_name="s")

  @functools.partial(
      pl.kernel,
      mesh=mesh,
      out_type=jax.ShapeDtypeStruct((BATCH, EMB), jnp.float32),
      compiler_params=pltpu.CompilerParams(use_tc_tiling_on_sc=False),
      scratch_types=[
          pltpu.VMEM((NCHUNK, CHUNK), jnp.int32),
          pltpu.VMEM((NCHUNK, CHUNK), jnp.int32),
          pltpu.VMEM((BPW, EMB), jnp.float32),
          pltpu.VMEM((BPW, EMB), jnp.float32),
          pltpu.SemaphoreType.DMA,
      ],
  )
  def k(x0_hbm, x1_hbm, e1_hbm, e2_hbm, out_hbm,
        idx1_v, idx2_v, rows1_v, rows2_v, sem):
    wid = lax.axis_index("s") * NUM_CORES + lax.axis_index("c")
    base = wid * BPW

    pltpu.sync_copy(x0_hbm.at[pl.ds(wid * NCHUNK, NCHUNK)], idx1_v)
    pltpu.sync_copy(x1_hbm.at[pl.ds(wid * NCHUNK, NCHUNK)], idx2_v)

    copies = []
    for c in range(NCHUNK):
      copies.append(pltpu.async_copy(
          e1_hbm.at[idx1_v.at[c]],
          rows1_v.at[pl.ds(c * CHUNK, CHUNK)], sem))
      copies.append(pltpu.async_copy(
          e2_hbm.at[idx2_v.at[c]],
          rows2_v.at[pl.ds(c * CHUNK, CHUNK)], sem))
    for cp in copies:
      cp.wait()

    def body(i, _):
      rows1_v[i, :] = rows1_v[i, :] * rows2_v[i, :]
      return 0
    lax.fori_loop(0, BPW, body, 0)

    pltpu.sync_copy(rows1_v, out_hbm.at[pl.ds(base, BPW)])

  return k(x0, x1, emb1, emb2)


def kernel(x, emb1, emb2):
  x0 = x[0].reshape(NW * NCHUNK, CHUNK)
  x1 = x[1].reshape(NW * NCHUNK, CHUNK)
  return _sc_interac(x0, x1, emb1, emb2)


# speedup vs baseline: 1.4931x; 1.4931x over previous
import functools

import jax
import jax.numpy as jnp
from jax import lax
from jax.experimental import pallas as pl
from jax.experimental.pallas import tpu as pltpu
from jax.experimental.pallas import tpu_sc as plsc

EMB = 16
BATCH = 16384
NUM_CORES = 2
NUM_SUBCORES = 16
NW = NUM_CORES * NUM_SUBCORES
BPW = BATCH // NW
HALF = BPW // 2
IDXW = 128
NIDX = BPW // IDXW


def _sc_interac(x0, x1, emb1, emb2):
  mesh = plsc.VectorSubcoreMesh(core_axis_name="c", subcore_axis_name="s")

  @functools.partial(
      pl.kernel,
      mesh=mesh,
      out_type=jax.ShapeDtypeStruct((BATCH, EMB), jnp.float32),
      scratch_types=[
          pltpu.VMEM((NIDX, IDXW), jnp.int32),
          pltpu.VMEM((NIDX, IDXW), jnp.int32),
          pltpu.VMEM((HALF, EMB), jnp.float32),
          pltpu.VMEM((HALF, EMB), jnp.float32),
          pltpu.SemaphoreType.DMA,
      ],
  )
  def k(x0_hbm, x1_hbm, e1_hbm, e2_hbm, out_hbm,
        idx1_v, idx2_v, rows1_v, rows2_v, sem):
    wid = lax.axis_index("s") * NUM_CORES + lax.axis_index("c")
    base = wid * BPW

    pltpu.sync_copy(x0_hbm.at[pl.ds(wid * NIDX, NIDX)], idx1_v)
    pltpu.sync_copy(x1_hbm.at[pl.ds(wid * NIDX, NIDX)], idx2_v)

    for h in range(2):
      for c in range(2 * h, 2 * h + 2):
        def fire(g, _, c=c):
          iv1 = idx1_v[c, pl.ds(g * EMB, EMB)]
          iv2 = idx2_v[c, pl.ds(g * EMB, EMB)]
          rbase = (c - 2 * h) * IDXW + g * EMB
          for l in range(EMB):
            pltpu.async_copy(e1_hbm.at[pl.ds(iv1[l], 1)],
                             rows1_v.at[pl.ds(rbase + l, 1)], sem)
            pltpu.async_copy(e2_hbm.at[pl.ds(iv2[l], 1)],
                             rows2_v.at[pl.ds(rbase + l, 1)], sem)
          return 0
        lax.fori_loop(0, IDXW // EMB, fire, 0)

      def drain(r, _):
        pltpu.make_async_copy(
            e1_hbm.at[pl.ds(0, 1)], rows1_v.at[pl.ds(0, 1)], sem).wait()
        pltpu.make_async_copy(
            e2_hbm.at[pl.ds(0, 1)], rows2_v.at[pl.ds(0, 1)], sem).wait()
        return 0
      lax.fori_loop(0, HALF, drain, 0)

      def mul(r, _):
        rows1_v[r, :] = rows1_v[r, :] * rows2_v[r, :]
        return 0
      lax.fori_loop(0, HALF, mul, 0)

      pltpu.sync_copy(rows1_v, out_hbm.at[pl.ds(base + h * HALF, HALF)])

  return k(x0, x1, emb1, emb2)


def kernel(x, emb1, emb2):
  x0 = x[0].reshape(NW * NIDX, IDXW)
  x1 = x[1].reshape(NW * NIDX, IDXW)
  return _sc_interac(x0, x1, emb1, emb2)
